# Initial kernel scaffold; baseline (speedup 1.0000x reference)
import jax
import jax.numpy as jnp
from jax.experimental import pallas as pl


def kernel(x, qkv_w, proj_w, proj_b):
    raise NotImplementedError("write your pallas kernel here")


# speedup vs baseline: 1.2214x; 1.2214x over previous
import jax
import jax.numpy as jnp
from jax import lax
from jax.experimental import pallas as pl
from jax.experimental.pallas import tpu as pltpu

KN = 16
NH = 8
R = 128
T = R // KN


def _knn_body(x_ref, idx_ref, *, nb, n, c):
    i = pl.program_id(1)
    X = x_ref[0]
    xb = x_ref[0, :, pl.ds(i * nb, nb)]
    dot = lax.dot_general(xb, X, (((0,), (0,)), ((), ())),
                          preferred_element_type=jnp.float32)
    inner = -2.0 * dot
    xx_row = jnp.sum(X * X, axis=0, keepdims=True)
    ones = jnp.ones((c, 1), dtype=jnp.float32)
    xx_col = lax.dot_general(xb * xb, ones, (((0,), (0,)), ((), ())),
                             preferred_element_type=jnp.float32)
    s = (-xx_col - inner) - xx_row
    iota_l = lax.broadcasted_iota(jnp.int32, (nb, n), 1)
    cols = []
    for _ in range(KN):
        m = jnp.max(s, axis=1, keepdims=True)
        a = jnp.min(jnp.where(s == m, iota_l, n), axis=1, keepdims=True)
        cols.append(a)
        s = jnp.where(iota_l == a, -jnp.inf, s)
    idx_ref[0] = jnp.concatenate(cols, axis=1)


def _attn_body(x_ref, idx_ref, qkvw_ref, projw_ref, projb_ref, out_ref,
               *, n, c):
    t = pl.program_id(1)
    X = x_ref[0]
    idxr = idx_ref[0, 0, 0]
    oh = (lax.broadcasted_iota(jnp.int32, (n, R), 0) == idxr
          ).astype(jnp.float32)
    gT = lax.dot_general(X, oh, (((1,), (0,)), ((), ())),
                         preferred_element_type=jnp.float32)
    cen = (lax.broadcasted_iota(jnp.int32, (n, T), 0) ==
           t * T + lax.broadcasted_iota(jnp.int32, (n, T), 1)
           ).astype(jnp.float32)
    Xc = lax.dot_general(X, cen, (((1,), (0,)), ((), ())),
                         preferred_element_type=jnp.float32)
    rep = (lax.broadcasted_iota(jnp.int32, (T, R), 1) // KN ==
           lax.broadcasted_iota(jnp.int32, (T, R), 0)).astype(jnp.float32)
    x_repT = lax.dot_general(Xc, rep, (((1,), (0,)), ((), ())),
                             preferred_element_type=jnp.float32)
    fT = gT - x_repT
    qkvT = lax.dot_general(qkvw_ref[...], fT, (((1,), (0,)), ((), ())),
                           preferred_element_type=jnp.float32)
    scale = (c // NH) ** -0.5
    hd = c // NH
    bm = (lax.broadcasted_iota(jnp.int32, (R, R), 0) // KN ==
          lax.broadcasted_iota(jnp.int32, (R, R), 1) // KN)
    outs = []
    for h in range(NH):
        qh = qkvT[h * hd:(h + 1) * hd]
        kh = qkvT[c + h * hd:c + (h + 1) * hd]
        vh = qkvT[2 * c + h * hd:2 * c + (h + 1) * hd]
        s = lax.dot_general(kh, qh, (((0,), (0,)), ((), ())),
                            preferred_element_type=jnp.float32)
        s = jnp.where(bm, s * scale, -jnp.inf)
        s = s - jnp.max(s, axis=0, keepdims=True)
        e = jnp.exp(s)
        p = e / jnp.sum(e, axis=0, keepdims=True)
        outs.append(lax.dot_general(vh, p, (((1,), (0,)), ((), ())),
                                    preferred_element_type=jnp.float32))
    ovall = jnp.stack(outs, axis=0)
    ov5 = ovall.reshape(NH, hd, R // KN, 2, 8)
    outT = jnp.transpose(ov5, (4, 1, 2, 0, 3)).reshape(c, R)
    oT = lax.dot_general(projw_ref[...], outT, (((1,), (0,)), ((), ())),
                         preferred_element_type=jnp.float32) + projb_ref[...]
    out_ref[0] = jnp.concatenate([oT, x_repT], axis=0)


def kernel(x, qkv_w, proj_w, proj_b):
    B, C, N = x.shape
    nb = min(256, N)
    import functools
    knn = pl.pallas_call(
        functools.partial(_knn_body, nb=nb, n=N, c=C),
        grid=(B, N // nb),
        in_specs=[pl.BlockSpec((1, C, N), lambda b, i: (b, 0, 0))],
        out_specs=pl.BlockSpec((1, nb, KN), lambda b, i: (b, i, 0)),
        out_shape=jax.ShapeDtypeStruct((B, N, KN), jnp.int32),
    )
    idx = knn(x)
    nblk = (N * KN) // R
    idx3 = idx.reshape(B, nblk, 1, R)
    pb2 = proj_b.reshape(C, 1)
    attn = pl.pallas_call(
        functools.partial(_attn_body, n=N, c=C),
        grid=(B, nblk),
        in_specs=[
            pl.BlockSpec((1, C, N), lambda b, i: (b, 0, 0)),
            pl.BlockSpec((1, 1, 1, R), lambda b, i: (b, i, 0, 0)),
            pl.BlockSpec((3 * C, C), lambda b, i: (0, 0)),
            pl.BlockSpec((C, C), lambda b, i: (0, 0)),
            pl.BlockSpec((C, 1), lambda b, i: (0, 0)),
        ],
        out_specs=pl.BlockSpec((1, 2 * C, R), lambda b, i: (b, 0, i)),
        out_shape=jax.ShapeDtypeStruct((B, 2 * C, N * KN), jnp.float32),
    )
    y = attn(x, idx3, qkv_w, proj_w, pb2)
    return y.reshape(B, 2 * C, N, KN)


# speedup vs baseline: 1.2261x; 1.0039x over previous
import jax
import jax.numpy as jnp
from jax import lax
from jax.experimental import pallas as pl
from jax.experimental.pallas import tpu as pltpu

KN = 16
NH = 8
R = 128
T = R // KN


def _knn_body(x_ref, idx_ref, *, nb, n, c):
    i = pl.program_id(1)
    X = x_ref[0]
    xb = x_ref[0, :, pl.ds(i * nb, nb)]
    dot = lax.dot_general(xb, X, (((0,), (0,)), ((), ())),
                          preferred_element_type=jnp.float32)
    inner = -2.0 * dot
    xx_row = jnp.sum(X * X, axis=0, keepdims=True)
    ones = jnp.ones((c, 1), dtype=jnp.float32)
    xx_col = lax.dot_general(xb * xb, ones, (((0,), (0,)), ((), ())),
                             preferred_element_type=jnp.float32)
    s = (-xx_col - inner) - xx_row
    iota_l = lax.broadcasted_iota(jnp.int32, (nb, n), 1)
    cols = []
    for _ in range(KN):
        m = jnp.max(s, axis=1, keepdims=True)
        a = jnp.min(jnp.where(s == m, iota_l, n), axis=1, keepdims=True)
        cols.append(a)
        s = jnp.where(iota_l == a, -jnp.inf, s)
    idx_ref[0] = jnp.concatenate(cols, axis=1)


def _attn_body(x_ref, idx_ref, qkvw_ref, projw_ref, projb_ref, out_ref,
               *, n, c):
    t = pl.program_id(1)
    X = x_ref[0]
    idxr = idx_ref[0, 0, 0]
    oh = (lax.broadcasted_iota(jnp.int32, (n, R), 0) == idxr
          ).astype(jnp.float32)
    gT = lax.dot_general(X, oh, (((1,), (0,)), ((), ())),
                         preferred_element_type=jnp.float32)
    cen = (lax.broadcasted_iota(jnp.int32, (n, T), 0) ==
           t * T + lax.broadcasted_iota(jnp.int32, (n, T), 1)
           ).astype(jnp.float32)
    Xc = lax.dot_general(X, cen, (((1,), (0,)), ((), ())),
                         preferred_element_type=jnp.float32)
    rep = (lax.broadcasted_iota(jnp.int32, (T, R), 1) // KN ==
           lax.broadcasted_iota(jnp.int32, (T, R), 0)).astype(jnp.float32)
    x_repT = lax.dot_general(Xc, rep, (((1,), (0,)), ((), ())),
                             preferred_element_type=jnp.float32)
    fT = gT - x_repT
    qkvT = lax.dot_general(qkvw_ref[...], fT, (((1,), (0,)), ((), ())),
                           preferred_element_type=jnp.float32)
    scale = (c // NH) ** -0.5
    hd = c // NH
    bm = (lax.broadcasted_iota(jnp.int32, (R, R), 0) // KN ==
          lax.broadcasted_iota(jnp.int32, (R, R), 1) // KN)
    outs = []
    for h in range(NH):
        qh = qkvT[h * hd:(h + 1) * hd]
        kh = qkvT[c + h * hd:c + (h + 1) * hd]
        vh = qkvT[2 * c + h * hd:2 * c + (h + 1) * hd]
        s = lax.dot_general(kh, qh, (((0,), (0,)), ((), ())),
                            preferred_element_type=jnp.float32)
        s = jnp.where(bm, s * scale, -jnp.inf)
        s = s - jnp.max(s, axis=0, keepdims=True)
        e = jnp.exp(s)
        p = e / jnp.sum(e, axis=0, keepdims=True)
        ovh = lax.dot_general(vh, p, (((1,), (0,)), ((), ())),
                              preferred_element_type=jnp.float32)
        ia = lax.broadcasted_iota(jnp.int32, (R, 2 * T), 0)
        ib = lax.broadcasted_iota(jnp.int32, (R, 2 * T), 1)
        blocks = []
        for il in range(8):
            sil = ((ia // KN == ib // 2) &
                   (ia % KN == (ib % 2) * 8 + il)).astype(jnp.float32)
            blocks.append(lax.dot_general(ovh, sil, (((1,), (0,)), ((), ())),
                                          preferred_element_type=jnp.float32))
        outs.append(jnp.concatenate(blocks, axis=0))
    yall = jnp.concatenate(outs, axis=1)
    pa = lax.broadcasted_iota(jnp.int32, (R, R), 0)
    pb = lax.broadcasted_iota(jnp.int32, (R, R), 1)
    perm = ((pa // KN == (pb % KN) // 2) & ((pa % KN) // 2 == pb // KN) &
            (pa % 2 == pb % 2)).astype(jnp.float32)
    outT = lax.dot_general(yall, perm, (((1,), (0,)), ((), ())),
                           preferred_element_type=jnp.float32)
    oT = lax.dot_general(projw_ref[...], outT, (((1,), (0,)), ((), ())),
                         preferred_element_type=jnp.float32) + projb_ref[...]
    out_ref[0] = jnp.concatenate([oT, x_repT], axis=0)


def kernel(x, qkv_w, proj_w, proj_b):
    B, C, N = x.shape
    nb = min(256, N)
    import functools
    knn = pl.pallas_call(
        functools.partial(_knn_body, nb=nb, n=N, c=C),
        grid=(B, N // nb),
        in_specs=[pl.BlockSpec((1, C, N), lambda b, i: (b, 0, 0))],
        out_specs=pl.BlockSpec((1, nb, KN), lambda b, i: (b, i, 0)),
        out_shape=jax.ShapeDtypeStruct((B, N, KN), jnp.int32),
    )
    idx = knn(x)
    nblk = (N * KN) // R
    idx3 = idx.reshape(B, nblk, 1, R)
    pb2 = proj_b.reshape(C, 1)
    attn = pl.pallas_call(
        functools.partial(_attn_body, n=N, c=C),
        grid=(B, nblk),
        in_specs=[
            pl.BlockSpec((1, C, N), lambda b, i: (b, 0, 0)),
            pl.BlockSpec((1, 1, 1, R), lambda b, i: (b, i, 0, 0)),
            pl.BlockSpec((3 * C, C), lambda b, i: (0, 0)),
            pl.BlockSpec((C, C), lambda b, i: (0, 0)),
            pl.BlockSpec((C, 1), lambda b, i: (0, 0)),
        ],
        out_specs=pl.BlockSpec((1, 2 * C, R), lambda b, i: (b, 0, i)),
        out_shape=jax.ShapeDtypeStruct((B, 2 * C, N * KN), jnp.float32),
    )
    y = attn(x, idx3, qkv_w, proj_w, pb2)
    return y.reshape(B, 2 * C, N, KN)
